# Initial kernel scaffold; baseline (speedup 1.0000x reference)
import jax
import jax.numpy as jnp
from jax.experimental import pallas as pl


def kernel(DPTD_name_1, DPTD_name_2, table):
    raise NotImplementedError("write your pallas kernel here")


# speedup vs baseline: 4.3132x; 4.3132x over previous
import functools

import jax
import jax.numpy as jnp
from jax import lax
from jax.experimental import pallas as pl
from jax.experimental.pallas import tpu as pltpu
from jax.experimental.pallas import tpu_sc as plsc

VOCAB = 1000
EMBED_DIM = 20
BATCH = 16384
HIST = 50

ROWS = BATCH * HIST
IDX_MINOR = 128
ROWS128 = ROWS // IDX_MINOR
NUM_WORKERS = 32
GROUPS_PER_W = ROWS128 // NUM_WORKERS
CHUNK = 8
ITERS = GROUPS_PER_W // CHUNK


def _body(idx1_hbm, idx2_hbm, table_hbm, out1_hbm, out2_hbm,
          idx_v, rows_v, gsem):
    wid = lax.axis_index("s") * 2 + lax.axis_index("c")
    base = wid * GROUPS_PER_W

    def phase(idx_hbm, out_hbm):
        def body(i, carry):
            off = base + i * CHUNK
            pltpu.sync_copy(idx_hbm.at[pl.ds(off, CHUNK)], idx_v)
            descs = [
                pltpu.async_copy(table_hbm.at[idx_v.at[j]], rows_v.at[j], gsem)
                for j in range(CHUNK)
            ]
            for d in descs:
                d.wait()
            pltpu.sync_copy(rows_v, out_hbm.at[pl.ds(off, CHUNK)])
            return carry
        lax.fori_loop(0, ITERS, body, 0)

    phase(idx1_hbm, out1_hbm)
    phase(idx2_hbm, out2_hbm)


def kernel(DPTD_name_1, DPTD_name_2, table):
    idx1 = DPTD_name_1.reshape(ROWS128, IDX_MINOR).astype(jnp.int32)
    idx2 = DPTD_name_2.reshape(ROWS128, IDX_MINOR).astype(jnp.int32)

    mesh = plsc.VectorSubcoreMesh(core_axis_name="c", subcore_axis_name="s")
    out_t = jax.ShapeDtypeStruct((ROWS128, IDX_MINOR, EMBED_DIM), jnp.float32)
    run = pl.kernel(
        _body,
        out_type=(out_t, out_t),
        mesh=mesh,
        scratch_types=[
            pltpu.VMEM((CHUNK, IDX_MINOR), jnp.int32),
            pltpu.VMEM((CHUNK, IDX_MINOR, EMBED_DIM), jnp.float32),
            pltpu.SemaphoreType.DMA,
        ],
        compiler_params=pltpu.CompilerParams(use_tc_tiling_on_sc=False),
    )
    out1, out2 = run(idx1, idx2, table)
    shape = (1, BATCH, HIST, EMBED_DIM)
    return (out1.reshape(shape), out2.reshape(shape))


# speedup vs baseline: 9.1672x; 2.1254x over previous
import jax
import jax.numpy as jnp
from jax import lax
from jax.experimental import pallas as pl
from jax.experimental.pallas import tpu as pltpu
from jax.experimental.pallas import tpu_sc as plsc

VOCAB = 1000
EMBED_DIM = 20
BATCH = 16384
HIST = 50

NUM_WORKERS = 32
BW = BATCH // NUM_WORKERS
NGROUPS = BW // 16


def _body(idx1_hbm, idx2_hbm, table_hbm, out1_hbm, out2_hbm,
          table_v, idx_v, stage_v, sem):
    wid = lax.axis_index("s") * 2 + lax.axis_index("c")
    b0 = wid * BW

    pltpu.sync_copy(table_hbm, table_v)

    def phase(idx_hbm, out_hbm):
        def h_body(h, carry):
            pltpu.sync_copy(idx_hbm.at[h, pl.ds(b0, BW)], idx_v)

            def g_body(g, c):
                iv = idx_v[pl.ds(g * 16, 16)]
                for d in range(EMBED_DIM):
                    dsplat = jnp.full((16,), d, jnp.int32)
                    vals = plsc.load_gather(table_v, [dsplat, iv])
                    stage_v[d, pl.ds(g * 16, 16)] = vals
                return c
            lax.fori_loop(0, NGROUPS, g_body, 0, unroll=True)

            pltpu.async_copy(
                stage_v,
                out_hbm.at[pl.ds(h * EMBED_DIM, EMBED_DIM), pl.ds(b0, BW)],
                sem,
            ).wait()
            return carry
        lax.fori_loop(0, HIST, h_body, 0)

    phase(idx1_hbm, out1_hbm)
    phase(idx2_hbm, out2_hbm)


def kernel(DPTD_name_1, DPTD_name_2, table):
    idx1_t = DPTD_name_1.astype(jnp.int32).T
    idx2_t = DPTD_name_2.astype(jnp.int32).T
    table_t = table.T

    mesh = plsc.VectorSubcoreMesh(
        core_axis_name="c", subcore_axis_name="s", num_cores=2,
        num_subcores=16)
    out_t = jax.ShapeDtypeStruct((HIST * EMBED_DIM, BATCH), jnp.float32)
    run = pl.kernel(
        _body,
        out_type=(out_t, out_t),
        mesh=mesh,
        scratch_types=[
            pltpu.VMEM((EMBED_DIM, VOCAB), jnp.float32),
            pltpu.VMEM((BW,), jnp.int32),
            pltpu.VMEM((EMBED_DIM, BW), jnp.float32),
            pltpu.SemaphoreType.DMA,
        ],
        compiler_params=pltpu.CompilerParams(
            use_tc_tiling_on_sc=False, needs_layout_passes=False),
    )
    o1, o2 = run(idx1_t, idx2_t, table_t)
    def to_logical(o):
        return o.reshape(HIST, EMBED_DIM, BATCH).transpose(2, 0, 1)[None]
    return (to_logical(o1), to_logical(o2))


# speedup vs baseline: 9.7193x; 1.0602x over previous
import jax
import jax.numpy as jnp
from jax import lax
from jax.experimental import pallas as pl
from jax.experimental.pallas import tpu as pltpu
from jax.experimental.pallas import tpu_sc as plsc

VOCAB = 1000
EMBED_DIM = 20
BATCH = 16384
HIST = 50

NUM_WORKERS = 32
BW = BATCH // NUM_WORKERS
NGROUPS = BW // 16


def _body(idx1_hbm, idx2_hbm, table_hbm, out1_hbm, out2_hbm,
          table_v, idx_v, stage_v, isem0, isem1, osem0, osem1):
    wid = lax.axis_index("s") * 2 + lax.axis_index("c")
    b0 = wid * BW
    isems = (isem0, isem1)
    osems = (osem0, osem1)

    pltpu.sync_copy(table_hbm, table_v)

    def idx_copy(idx_hbm, h, p):
        return pltpu.make_async_copy(
            idx_hbm.at[h, pl.ds(b0, BW)], idx_v.at[p], isems[p])

    def out_copy(out_hbm, h, p):
        return pltpu.make_async_copy(
            stage_v.at[p],
            out_hbm.at[pl.ds(h * EMBED_DIM, EMBED_DIM), pl.ds(b0, BW)],
            osems[p])

    def phase(idx_hbm, out_hbm):
        idx_copy(idx_hbm, 0, 0).start()

        def step(i, h, p):
            idx_copy(idx_hbm, h, p).wait()

            @pl.when(h + 1 < HIST)
            def _pf():
                idx_copy(idx_hbm, h + 1, 1 - p).start()

            @pl.when(i > 0)
            def _wo():
                out_copy(out_hbm, h, p).wait()

            def g_body(g, c):
                iv = idx_v[p, pl.ds(g * 16, 16)]
                for d in range(EMBED_DIM):
                    dsplat = jnp.full((16,), d, jnp.int32)
                    vals = plsc.load_gather(table_v, [dsplat, iv])
                    stage_v[p, d, pl.ds(g * 16, 16)] = vals
                return c
            lax.fori_loop(0, NGROUPS, g_body, 0, unroll=True)

            out_copy(out_hbm, h, p).start()

        def pair(i, carry):
            step(i, 2 * i, 0)
            step(i, 2 * i + 1, 1)
            return carry
        lax.fori_loop(0, HIST // 2, pair, 0)

        out_copy(out_hbm, HIST - 2, 0).wait()
        out_copy(out_hbm, HIST - 1, 1).wait()

    phase(idx1_hbm, out1_hbm)
    phase(idx2_hbm, out2_hbm)


def kernel(DPTD_name_1, DPTD_name_2, table):
    idx1_t = DPTD_name_1.astype(jnp.int32).T
    idx2_t = DPTD_name_2.astype(jnp.int32).T
    table_t = table.T

    mesh = plsc.VectorSubcoreMesh(
        core_axis_name="c", subcore_axis_name="s", num_cores=2,
        num_subcores=16)
    out_t = jax.ShapeDtypeStruct((HIST * EMBED_DIM, BATCH), jnp.float32)
    run = pl.kernel(
        _body,
        out_type=(out_t, out_t),
        mesh=mesh,
        scratch_types=[
            pltpu.VMEM((EMBED_DIM, VOCAB), jnp.float32),
            pltpu.VMEM((2, BW), jnp.int32),
            pltpu.VMEM((2, EMBED_DIM, BW), jnp.float32),
            pltpu.SemaphoreType.DMA,
            pltpu.SemaphoreType.DMA,
            pltpu.SemaphoreType.DMA,
            pltpu.SemaphoreType.DMA,
        ],
        compiler_params=pltpu.CompilerParams(
            use_tc_tiling_on_sc=False, needs_layout_passes=False),
    )
    o1, o2 = run(idx1_t, idx2_t, table_t)
    def to_logical(o):
        return o.reshape(HIST, EMBED_DIM, BATCH).transpose(2, 0, 1)[None]
    return (to_logical(o1), to_logical(o2))
